# baseline (speedup 1.0000x reference)
import functools

import jax
import jax.numpy as jnp
from jax import lax
from jax.experimental import pallas as pl
from jax.experimental.pallas import tpu as pltpu
from jax.experimental.pallas import tpu_sc as plsc

B = 4096
L = 50
D = 64
V = 21000
NUM_LT = 6
NUM_GT = 5
NUM_BAG = NUM_LT + NUM_GT
NUM_TAB = NUM_BAG + 1
NC, NS = 2, 16
NW = NC * NS
BW = B // NW
CH = 16
NCH = BW // CH
ROWS = CH * L


def _sc_group(tab_s, ids_flat, nbag, show_tab=None, show_ids=None):
    ntab = nbag + (0 if show_tab is None else 1)
    extra = () if show_tab is None else (show_tab, show_ids)
    mesh = plsc.VectorSubcoreMesh(
        core_axis_name="c", subcore_axis_name="s", num_cores=NC, num_subcores=NS
    )

    @functools.partial(
        pl.kernel,
        out_type=jax.ShapeDtypeStruct((B, ntab * D), jnp.bfloat16),
        mesh=mesh,
        scratch_types=[
            pltpu.VMEM((BW * L,), jnp.int32),
            pltpu.VMEM((ROWS, D), jnp.bfloat16),
            pltpu.VMEM((ROWS, D), jnp.bfloat16),
            pltpu.VMEM((BW, D), jnp.bfloat16),
            pltpu.SemaphoreType.DMA,
            pltpu.SemaphoreType.DMA,
        ],
        compiler_params=pltpu.CompilerParams(use_tc_tiling_on_sc=False),
    )
    def k(*refs):
        if show_tab is None:
            tab, ids, out = refs[:3]
            idx_all, buf_a, buf_b, acc_v, sem_a, sem_b = refs[3:]
            show_t = show_i = None
        else:
            tab, ids, show_t, show_i, out = refs[:5]
            idx_all, buf_a, buf_b, acc_v, sem_a, sem_b = refs[5:]
        wid = lax.axis_index("s") * NC + lax.axis_index("c")
        base = wid * BW

        def gather_start(tab, c, buf, sem):
            pltpu.async_copy(tab.at[idx_all.at[pl.ds(c * ROWS, ROWS)]],
                             buf, sem)

        def gather_wait(tab, c, buf, sem):
            pltpu.make_async_copy(tab.at[idx_all.at[pl.ds(c * ROWS, ROWS)]],
                                  buf, sem).wait()

        def compute_chunk(c, buf):
            def bag_body(j, _):
                row0 = j * L
                accs = tuple(buf[row0, pl.ds(32 * q, 32)] for q in range(2))

                def l_body(i, a):
                    r = row0 + 1 + 2 * i
                    a = tuple(jnp.maximum(a[q], buf[r, pl.ds(32 * q, 32)])
                              for q in range(2))
                    return tuple(jnp.maximum(a[q], buf[r + 1, pl.ds(32 * q, 32)])
                                 for q in range(2))

                accs = lax.fori_loop(0, (L - 2) // 2, l_body, accs)
                accs = tuple(jnp.maximum(accs[q],
                                         buf[row0 + L - 1, pl.ds(32 * q, 32)])
                             for q in range(2))
                for q in range(2):
                    acc_v[c * CH + j, pl.ds(32 * q, 32)] = accs[q]
                return 0

            lax.fori_loop(0, CH, bag_body, 0)

        for t in range(nbag):
            pltpu.sync_copy(ids.at[pl.ds(t * B * L + base * L, BW * L)],
                            idx_all)
            gather_start(tab, 0, buf_a, sem_a)
            gather_start(tab, 1, buf_b, sem_b)

            def pipe(i, _, tab=tab):
                for p, (buf, sem) in enumerate(((buf_a, sem_a), (buf_b, sem_b))):
                    c = 2 * i + p
                    gather_wait(tab, c, buf, sem)
                    compute_chunk(c, buf)

                    @pl.when(c + 2 < NCH)
                    def _(c=c, buf=buf, sem=sem, tab=tab):
                        gather_start(tab, c + 2, buf, sem)
                return 0

            lax.fori_loop(0, NCH // 2, pipe, 0)
            pltpu.sync_copy(acc_v, out.at[pl.ds(base, BW), pl.ds(t * D, D)])

        if show_tab is not None:
            pltpu.sync_copy(show_i.at[pl.ds(base, BW)],
                            idx_all.at[pl.ds(0, BW)])
            pltpu.async_copy(show_t.at[idx_all.at[pl.ds(0, BW)]], acc_v,
                             sem_a).wait()
            pltpu.sync_copy(acc_v,
                            out.at[pl.ds(base, BW), pl.ds(nbag * D, D)])

    return k(tab_s, ids_flat, *extra)


def _tc_heads(embs, lin_W, lin_b):
    def mm(e0_ref, e1_ref, e2_ref, e3_ref, e4_ref, e5_ref,
           w_ref, b_ref, out_ref):
        x = jnp.concatenate(
            [e0_ref[...], e1_ref[...], e2_ref[...], e3_ref[...],
             e4_ref[...], e5_ref[...]],
            axis=1).astype(jnp.float32)
        for i in range(lin_W.shape[0]):
            out_ref[i] = (
                jnp.dot(x, w_ref[i], preferred_element_type=jnp.float32)
                + b_ref[i][None, :]
            )

    return pl.pallas_call(
        mm,
        out_shape=jax.ShapeDtypeStruct((lin_W.shape[0], B, 5), jnp.float32),
    )(*embs, lin_W, lin_b)


def kernel(lt_ids_0, lt_ids_1, lt_ids_2, lt_ids_3, lt_ids_4, lt_ids_5,
           gt_ids_0, gt_ids_1, gt_ids_2, gt_ids_3, gt_ids_4,
           show_ids, lt_tables, gt_tables, show_table, lin_W, lin_b):
    lt_ids = [lt_ids_0, lt_ids_1, lt_ids_2, lt_ids_3, lt_ids_4, lt_ids_5]
    gt_ids = [gt_ids_0, gt_ids_1, gt_ids_2, gt_ids_3, gt_ids_4]
    off3 = (jnp.arange(3, dtype=jnp.int32) * V)[:, None]
    off2 = (jnp.arange(2, dtype=jnp.int32) * V)[:, None]

    def stack_ids(idl, off):
        ids = jnp.stack(idl).reshape(len(idl), B * L)
        return (ids + off).reshape(-1)

    embs = [
        _sc_group(
            lt_tables[0:2].astype(jnp.bfloat16).reshape(2 * V, D),
            stack_ids(lt_ids[0:2], off2), 2),
        _sc_group(
            lt_tables[2:4].astype(jnp.bfloat16).reshape(2 * V, D),
            stack_ids(lt_ids[2:4], off2), 2),
        _sc_group(
            lt_tables[4:6].astype(jnp.bfloat16).reshape(2 * V, D),
            stack_ids(lt_ids[4:6], off2), 2),
        _sc_group(
            gt_tables[0:2].astype(jnp.bfloat16).reshape(2 * V, D),
            stack_ids(gt_ids[0:2], off2), 2),
        _sc_group(
            gt_tables[2:4].astype(jnp.bfloat16).reshape(2 * V, D),
            stack_ids(gt_ids[2:4], off2), 2),
        _sc_group(
            gt_tables[4:5].astype(jnp.bfloat16).reshape(V, D),
            gt_ids[4].reshape(-1), 1,
            show_table.astype(jnp.bfloat16), show_ids),
    ]
    return _tc_heads(embs, lin_W, lin_b)


# speedup vs baseline: 1.0401x; 1.0401x over previous
import functools

import jax
import jax.numpy as jnp
from jax import lax
from jax.experimental import pallas as pl
from jax.experimental.pallas import tpu as pltpu
from jax.experimental.pallas import tpu_sc as plsc

B = 4096
L = 50
D = 64
V = 21000
NUM_LT = 6
NUM_GT = 5
NUM_BAG = NUM_LT + NUM_GT
NUM_TAB = NUM_BAG + 1
NC, NS = 2, 16
NW = NC * NS
BW = B // NW
CH = 16
NCH = BW // CH
ROWS = CH * L


def _sc_group(tab_s, ids_flat, nbag, show_tab=None, show_ids=None):
    ntab = nbag + (0 if show_tab is None else 1)
    extra = () if show_tab is None else (show_tab, show_ids)
    mesh = plsc.VectorSubcoreMesh(
        core_axis_name="c", subcore_axis_name="s", num_cores=NC, num_subcores=NS
    )

    @functools.partial(
        pl.kernel,
        out_type=jax.ShapeDtypeStruct((B, ntab * D), jnp.bfloat16),
        mesh=mesh,
        scratch_types=[
            pltpu.VMEM((BW * L,), jnp.int32),
            pltpu.VMEM((ROWS, D), jnp.bfloat16),
            pltpu.VMEM((ROWS, D), jnp.bfloat16),
            pltpu.VMEM((BW, D), jnp.bfloat16),
            pltpu.SemaphoreType.DMA,
            pltpu.SemaphoreType.DMA,
        ],
        compiler_params=pltpu.CompilerParams(use_tc_tiling_on_sc=False),
    )
    def k(*refs):
        if show_tab is None:
            tab, ids, out = refs[:3]
            idx_all, buf_a, buf_b, acc_v, sem_a, sem_b = refs[3:]
            show_t = show_i = None
        else:
            tab, ids, show_t, show_i, out = refs[:5]
            idx_all, buf_a, buf_b, acc_v, sem_a, sem_b = refs[5:]
        wid = lax.axis_index("s") * NC + lax.axis_index("c")
        base = wid * BW

        def gather_start(tab, c, buf, sem):
            pltpu.async_copy(tab.at[idx_all.at[pl.ds(c * ROWS, ROWS)]],
                             buf, sem)

        def gather_wait(tab, c, buf, sem):
            pltpu.make_async_copy(tab.at[idx_all.at[pl.ds(c * ROWS, ROWS)]],
                                  buf, sem).wait()

        def compute_chunk(c, buf):
            def bag_body(j, _):
                row0 = j * L
                accs = tuple(buf[row0, pl.ds(32 * q, 32)] for q in range(2))

                def l_body(i, a):
                    r = row0 + 1 + 2 * i
                    a = tuple(jnp.maximum(a[q], buf[r, pl.ds(32 * q, 32)])
                              for q in range(2))
                    return tuple(jnp.maximum(a[q], buf[r + 1, pl.ds(32 * q, 32)])
                                 for q in range(2))

                accs = lax.fori_loop(0, (L - 2) // 2, l_body, accs)
                accs = tuple(jnp.maximum(accs[q],
                                         buf[row0 + L - 1, pl.ds(32 * q, 32)])
                             for q in range(2))
                for q in range(2):
                    acc_v[c * CH + j, pl.ds(32 * q, 32)] = accs[q]
                return 0

            lax.fori_loop(0, CH, bag_body, 0)

        for t in range(nbag):
            pltpu.sync_copy(ids.at[pl.ds(t * B * L + base * L, BW * L)],
                            idx_all)
            gather_start(tab, 0, buf_a, sem_a)
            gather_start(tab, 1, buf_b, sem_b)

            def pipe(i, _, tab=tab):
                for p, (buf, sem) in enumerate(((buf_a, sem_a), (buf_b, sem_b))):
                    c = 2 * i + p
                    gather_wait(tab, c, buf, sem)
                    compute_chunk(c, buf)

                    @pl.when(c + 2 < NCH)
                    def _(c=c, buf=buf, sem=sem, tab=tab):
                        gather_start(tab, c + 2, buf, sem)
                return 0

            lax.fori_loop(0, NCH // 2, pipe, 0)
            pltpu.sync_copy(acc_v, out.at[pl.ds(base, BW), pl.ds(t * D, D)])

        if show_tab is not None:
            pltpu.sync_copy(show_i.at[pl.ds(base, BW)],
                            idx_all.at[pl.ds(0, BW)])
            pltpu.async_copy(show_t.at[idx_all.at[pl.ds(0, BW)]], acc_v,
                             sem_a).wait()
            pltpu.sync_copy(acc_v,
                            out.at[pl.ds(base, BW), pl.ds(nbag * D, D)])

    return k(tab_s, ids_flat, *extra)


def _tc_heads(embs, lin_W, lin_b):
    def mm(e0_ref, e1_ref, e2_ref, e3_ref, w_ref, b_ref, out_ref):
        x = jnp.concatenate(
            [e0_ref[...], e1_ref[...], e2_ref[...], e3_ref[...]],
            axis=1).astype(jnp.float32)
        for i in range(lin_W.shape[0]):
            out_ref[i] = (
                jnp.dot(x, w_ref[i], preferred_element_type=jnp.float32)
                + b_ref[i][None, :]
            )

    return pl.pallas_call(
        mm,
        out_shape=jax.ShapeDtypeStruct((lin_W.shape[0], B, 5), jnp.float32),
    )(*embs, lin_W, lin_b)


def kernel(lt_ids_0, lt_ids_1, lt_ids_2, lt_ids_3, lt_ids_4, lt_ids_5,
           gt_ids_0, gt_ids_1, gt_ids_2, gt_ids_3, gt_ids_4,
           show_ids, lt_tables, gt_tables, show_table, lin_W, lin_b):
    lt_ids = [lt_ids_0, lt_ids_1, lt_ids_2, lt_ids_3, lt_ids_4, lt_ids_5]
    gt_ids = [gt_ids_0, gt_ids_1, gt_ids_2, gt_ids_3, gt_ids_4]
    off3 = (jnp.arange(3, dtype=jnp.int32) * V)[:, None]
    off2 = (jnp.arange(2, dtype=jnp.int32) * V)[:, None]

    def stack_ids(idl, off):
        ids = jnp.stack(idl).reshape(len(idl), B * L)
        return (ids + off).reshape(-1)

    embs = [
        _sc_group(
            lt_tables[0:3].astype(jnp.bfloat16).reshape(3 * V, D),
            stack_ids(lt_ids[0:3], off3), 3),
        _sc_group(
            lt_tables[3:6].astype(jnp.bfloat16).reshape(3 * V, D),
            stack_ids(lt_ids[3:6], off3), 3),
        _sc_group(
            gt_tables[0:3].astype(jnp.bfloat16).reshape(3 * V, D),
            stack_ids(gt_ids[0:3], off3), 3),
        _sc_group(
            gt_tables[3:5].astype(jnp.bfloat16).reshape(2 * V, D),
            stack_ids(gt_ids[3:5], off2), 2,
            show_table.astype(jnp.bfloat16), show_ids),
    ]
    return _tc_heads(embs, lin_W, lin_b)
